# Initial kernel scaffold; baseline (speedup 1.0000x reference)
import jax
import jax.numpy as jnp
from jax.experimental import pallas as pl


def kernel(node_x, edge_index, edge_x, node_edge_index, node_edge_scatter_index, Wc, bc, Wn, bn, We, be):
    raise NotImplementedError("write your pallas kernel here")


# speedup vs baseline: 5.1466x; 5.1466x over previous
import functools

import jax
import jax.numpy as jnp
from jax import lax
from jax.experimental import pallas as pl
from jax.experimental.pallas import tpu as pltpu
from jax.experimental.pallas import tpu_sc as plsc

N_NODES = 10000
N_EDGES = 320000
D_NODE = 128
D_EDGE = 16
D_OUT = 128

NPAD = 10240
CH = 128
NCHUNK = N_EDGES // CH
NC = 2
NS = 16
NW = NC * NS
ITERS = (NCHUNK + NW - 1) // NW
ROWS_PER_TILE = NPAD // NS

def _sc_body(row_h, col_h, nei_h, scat_h, node_x_h, edge_x_h,
             accn_out, acce_out,
             idx_r, idx_c, idx_n, idx_s, rows_v, erows_v,
             acc_n, acc_e, sem1, sem2):
    cid = lax.axis_index("c")
    sid = lax.axis_index("s")
    w = sid * NC + cid

    _ZERO16 = jnp.zeros((16,), jnp.float32)

    def _zero_rows(i, _):
        for k in range(D_NODE // 16):
            rows_v[i, pl.ds(k * 16, 16)] = _ZERO16
        erows_v[i, pl.ds(0, 16)] = _ZERO16
        return 0
    lax.fori_loop(0, CH, _zero_rows, 0)
    base = sid * ROWS_PER_TILE
    for j in range(ROWS_PER_TILE // CH):
        pltpu.sync_copy(rows_v, acc_n.at[pl.ds(base + j * CH, CH)])
        pltpu.sync_copy(erows_v, acc_e.at[pl.ds(base + j * CH, CH)])
    plsc.subcore_barrier()

    def _chunk(i, _):
        c = w + i * NW

        @pl.when(c < NCHUNK)
        def _():
            b = c * CH
            pltpu.sync_copy(row_h.at[pl.ds(b, CH)], idx_r)
            pltpu.sync_copy(col_h.at[pl.ds(b, CH)], idx_c)
            pltpu.sync_copy(nei_h.at[pl.ds(b, CH)], idx_n)
            pltpu.sync_copy(scat_h.at[pl.ds(b, CH)], idx_s)
            cp1 = pltpu.async_copy(node_x_h.at[idx_r], rows_v, sem1)
            cp2 = pltpu.async_copy(edge_x_h.at[idx_n], erows_v, sem2)
            cp1.wait()
            cp2.wait()
            pltpu.sync_copy(rows_v, acc_n.at[idx_c], add=True)
            pltpu.sync_copy(erows_v, acc_e.at[idx_s], add=True)
        return 0

    lax.fori_loop(0, ITERS, _chunk, 0)
    plsc.subcore_barrier()

    for j in range(ROWS_PER_TILE // CH):
        r = base + j * CH
        pltpu.sync_copy(acc_n.at[pl.ds(r, CH)], accn_out.at[cid, pl.ds(r, CH)])
        pltpu.sync_copy(acc_e.at[pl.ds(r, CH)], acce_out.at[cid, pl.ds(r, CH)])


_sc_aggregate = functools.partial(
    pl.kernel,
    out_type=(
        jax.ShapeDtypeStruct((NC, NPAD, D_NODE), jnp.float32),
        jax.ShapeDtypeStruct((NC, NPAD, D_EDGE), jnp.float32),
    ),
    mesh=plsc.VectorSubcoreMesh(core_axis_name="c", subcore_axis_name="s"),
    scratch_types=[
        pltpu.VMEM((CH,), jnp.int32),
        pltpu.VMEM((CH,), jnp.int32),
        pltpu.VMEM((CH,), jnp.int32),
        pltpu.VMEM((CH,), jnp.int32),
        pltpu.VMEM((CH, D_NODE), jnp.float32),
        pltpu.VMEM((CH, D_EDGE), jnp.float32),
        pltpu.VMEM_SHARED((NPAD, D_NODE), jnp.float32),
        pltpu.VMEM_SHARED((NPAD, D_EDGE), jnp.float32),
        pltpu.SemaphoreType.DMA,
        pltpu.SemaphoreType.DMA,
    ],
    compiler_params=pltpu.CompilerParams(use_tc_tiling_on_sc=False),
)(_sc_body)


ROWS_BLK = 1000


def _tc_body(nx_ref, ap_ref0, ap_ref1, ep_ref0, ep_ref1,
             wct_ref, wnt_ref, wet_ref, b_ref, o_ref):
    x = nx_ref[...]
    a = ap_ref0[0] + ap_ref1[0]
    e = ep_ref0[0] + ep_ref1[0]
    out = (jnp.dot(x, wct_ref[...], preferred_element_type=jnp.float32)
           + jnp.dot(a, wnt_ref[...], preferred_element_type=jnp.float32)
           + jnp.dot(e, wet_ref[...], preferred_element_type=jnp.float32)
           + b_ref[...])
    nrm = jnp.sqrt(jnp.sum(out * out, axis=1, keepdims=True))
    out = out / jnp.maximum(nrm, 1e-12)
    o_ref[...] = jnp.where(out >= 0, out, 0.01 * out)


def _tc_update(node_x, aggr_p, aggr1_p, wct, wnt, wet, bias):
    grid = N_NODES // ROWS_BLK
    return pl.pallas_call(
        _tc_body,
        grid=(grid,),
        in_specs=[
            pl.BlockSpec((ROWS_BLK, D_NODE), lambda i: (i, 0)),
            pl.BlockSpec((1, ROWS_BLK, D_NODE), lambda i: (0, i, 0)),
            pl.BlockSpec((1, ROWS_BLK, D_NODE), lambda i: (1, i, 0)),
            pl.BlockSpec((1, ROWS_BLK, D_EDGE), lambda i: (0, i, 0)),
            pl.BlockSpec((1, ROWS_BLK, D_EDGE), lambda i: (1, i, 0)),
            pl.BlockSpec((D_NODE, D_OUT), lambda i: (0, 0)),
            pl.BlockSpec((D_NODE, D_OUT), lambda i: (0, 0)),
            pl.BlockSpec((D_EDGE, D_OUT), lambda i: (0, 0)),
            pl.BlockSpec((1, D_OUT), lambda i: (0, 0)),
        ],
        out_specs=pl.BlockSpec((ROWS_BLK, D_OUT), lambda i: (i, 0)),
        out_shape=jax.ShapeDtypeStruct((N_NODES, D_OUT), jnp.float32),
    )(node_x, aggr_p, aggr_p, aggr1_p, aggr1_p, wct, wnt, wet, bias)


def kernel(node_x, edge_index, edge_x, node_edge_index,
           node_edge_scatter_index, Wc, bc, Wn, bn, We, be):
    row = edge_index[0]
    col = edge_index[1]
    aggr_p, aggr1_p = _sc_aggregate(
        row, col, node_edge_index, node_edge_scatter_index, node_x, edge_x)
    bias = (bc + bn + be).reshape(1, D_OUT)
    return _tc_update(node_x, aggr_p, aggr1_p, Wc.T, Wn.T, We.T, bias)
